# baseline (speedup 1.0000x reference)
import functools
import math

import jax
import jax.numpy as jnp
from jax import lax
from jax.experimental import pallas as pl
from jax.experimental.pallas import tpu as pltpu
from jax.experimental.pallas import tpu_sc as plsc

_N = 10000
_D = 384
_DEG = 16
_K = 8
_H = 128
_STEPS = 3

_NT = 16
_CPT = 640
_NP = _NT * _CPT
_CH = 128
_NCH = _CPT // _CH
_LANES = 16
_VPT = _CPT * _K // _LANES

_RB = 2000
_NB = _N // _RB


def _coin_path_body(x_ref, q_ref, cw1_ref, cb1_ref, cw2_ref, cb2_ref,
                    wpe_ref, wpq_ref, pb1_ref, amps_ref, p_ref):
    x = x_ref[...]
    h = jnp.dot(x, cw1_ref[:_D, :], preferred_element_type=jnp.float32)
    hq = jnp.dot(q_ref[...], cw1_ref[_D:, :], preferred_element_type=jnp.float32)
    h = jnp.maximum(h + hq + cb1_ref[...], 0.0)
    amps_ref[...] = (
        jnp.dot(h, cw2_ref[...], preferred_element_type=jnp.float32)
        + cb2_ref[...]
    )
    p = jnp.dot(x, wpe_ref[...], preferred_element_type=jnp.float32)
    pq = jnp.dot(q_ref[...], wpq_ref[...], preferred_element_type=jnp.float32)
    p_ref[...] = p + pq + pb1_ref[...]


def _final_body(p_ref, u_ref, part_ref, ws_ref, pw2_ref, pb2_ref, out_ref):
    ss = jnp.sum(part_ref[...])
    scale = jnp.where(ss > 0.0, lax.rsqrt(ss), 1.0)
    us = jnp.dot(u_ref[...], ws_ref[...], preferred_element_type=jnp.float32)
    hp = jnp.maximum(p_ref[...] + us * scale, 0.0)
    out_ref[...] = (
        jnp.dot(hp, pw2_ref[...], preferred_element_type=jnp.float32)
        + pb2_ref[...]
    )


_S0 = 1.0 / math.sqrt(_N * _K)


def _walk_body(amps_hbm, nbr_hbm, u_hbm, part_hbm,
               shared_v, idx_v, amps_v, nbr_v, v_v, acc_v, ss_v, sem):
    t = lax.axis_index("s")
    base = t * _CPT
    last = _NT - 1
    tail = _N - last * _CPT
    iota = lax.iota(jnp.int32, _LANES)
    zeros = jnp.zeros((_LANES,), jnp.float32)

    pltpu.sync_copy(amps_hbm.at[pl.ds(base, _CPT)], amps_v)

    @pl.when(t < last)
    def _():
        pltpu.sync_copy(nbr_hbm.at[pl.ds(base, _CPT)], nbr_v)

    @pl.when(t == last)
    def _():
        pltpu.sync_copy(nbr_hbm.at[pl.ds(base, tail)], nbr_v.at[pl.ds(0, tail)])
        def zbody(i, carry):
            p = i * _LANES + iota
            plsc.store_scatter(
                amps_v,
                [lax.shift_right_logical(p, 3), lax.bitwise_and(p, 7)],
                zeros,
            )
            return carry
        lax.fori_loop(tail * _K // _LANES, _VPT, zbody, 0)

    limit = jnp.where(t == last, tail, _CPT)

    def ibody(g, carry):
        j = lax.div(g, _NCH)
        c = lax.rem(g, _NCH)
        for li in range(_CH // _LANES):
            rows = c * _CH + li * _LANES + iota
            vals = plsc.load_gather(nbr_v, [rows, jnp.full((_LANES,), 0, jnp.int32) + j])
            vals = jnp.where(rows < limit, vals, _NP - 1)
            plsc.store_scatter(
                idx_v,
                [jnp.full((_LANES,), 0, jnp.int32) + g, li * _LANES + iota],
                vals,
            )
        return carry
    lax.fori_loop(0, _DEG * _NCH, ibody, 0)

    def vmul(first):
        def body(i, carry):
            p = i * _LANES + iota
            r = lax.shift_right_logical(p, 3)
            c = lax.bitwise_and(p, 7)
            m = plsc.load_gather(amps_v, [r, c])
            if first:
                val = m * _S0
            else:
                a = plsc.load_gather(acc_v, [r, c])
                val = a * m
            plsc.store_scatter(v_v, [r, c], val)
            plsc.store_scatter(acc_v, [r, c], val)
            return carry
        lax.fori_loop(0, _VPT, body, 0, unroll=4)

    for step in range(_STEPS):
        vmul(step == 0)
        pltpu.sync_copy(v_v, shared_v.at[pl.ds(base, _CPT)])
        plsc.subcore_barrier()
        descs = []
        for g in range(_DEG * _NCH):
            ch = g % _NCH
            descs.append(
                pltpu.async_copy(
                    shared_v.at[idx_v.at[g]],
                    acc_v.at[pl.ds(ch * _CH, _CH)],
                    sem,
                    add=True,
                )
            )
        for d in descs:
            d.wait()
        plsc.subcore_barrier()

    pltpu.sync_copy(acc_v, u_hbm.at[t])

    def ss_body(i, s):
        p = i * _LANES + iota
        a = plsc.load_gather(acc_v, [lax.shift_right_logical(p, 3),
                                     lax.bitwise_and(p, 7)])
        return s + a * a
    ss = lax.fori_loop(0, _VPT, ss_body, jnp.zeros((_LANES,), jnp.float32),
                       unroll=4)
    ss_v[...] = ss
    pltpu.sync_copy(ss_v, part_hbm.at[t])


_walk = functools.partial(
    pl.kernel,
    out_type=(
        jax.ShapeDtypeStruct((_NT, _CPT, _K), jnp.float32),
        jax.ShapeDtypeStruct((_NT, _LANES), jnp.float32),
    ),
    mesh=plsc.VectorSubcoreMesh(
        core_axis_name="c", subcore_axis_name="s", num_cores=1
    ),
    compiler_params=pltpu.CompilerParams(
        needs_layout_passes=False, use_tc_tiling_on_sc=False
    ),
    scratch_types=[
        pltpu.VMEM_SHARED((_NP, _K), jnp.float32),
        pltpu.VMEM((_DEG * _NCH, _CH), jnp.int32),
        pltpu.VMEM((_CPT, _K), jnp.float32),
        pltpu.VMEM((_CPT, _DEG), jnp.int32),
        pltpu.VMEM((_CPT, _K), jnp.float32),
        pltpu.VMEM((_CPT, _K), jnp.float32),
        pltpu.VMEM((_LANES,), jnp.float32),
        pltpu.SemaphoreType.DMA,
    ],
)(_walk_body)


def kernel(emb_ex, q_emb, neighbors, coin_w1, coin_b1, coin_w2, coin_b2,
           path_w1, path_b1, path_w2, path_b2):
    n, d = emb_ex.shape
    k = coin_w2.shape[1]
    h = coin_w1.shape[1]

    q2 = q_emb[None, :]
    wpe = path_w1[:d]
    ws = path_w1[d:d + k]
    wpq = path_w1[d + k:]

    amps, p_part = pl.pallas_call(
        _coin_path_body,
        grid=(_NB,),
        in_specs=[
            pl.BlockSpec((_RB, d), lambda i: (i, 0)),
            pl.BlockSpec((1, d), lambda i: (0, 0)),
            pl.BlockSpec((2 * d, h), lambda i: (0, 0)),
            pl.BlockSpec((1, h), lambda i: (0, 0)),
            pl.BlockSpec((h, k), lambda i: (0, 0)),
            pl.BlockSpec((1, k), lambda i: (0, 0)),
            pl.BlockSpec((d, h), lambda i: (0, 0)),
            pl.BlockSpec((d, h), lambda i: (0, 0)),
            pl.BlockSpec((1, h), lambda i: (0, 0)),
        ],
        out_specs=[
            pl.BlockSpec((_RB, k), lambda i: (i, 0)),
            pl.BlockSpec((_RB, h), lambda i: (i, 0)),
        ],
        out_shape=[
            jax.ShapeDtypeStruct((_NP, k), jnp.float32),
            jax.ShapeDtypeStruct((n, h), jnp.float32),
        ],
    )(emb_ex, q2, coin_w1, coin_b1[None, :], coin_w2, coin_b2[None, :],
      wpe, wpq, path_b1[None, :])

    u3, part = _walk(amps, neighbors.astype(jnp.int32))
    u = u3.reshape(_NP, k)

    logits2 = pl.pallas_call(
        _final_body,
        grid=(_NB,),
        in_specs=[
            pl.BlockSpec((_RB, h), lambda i: (i, 0)),
            pl.BlockSpec((_RB, k), lambda i: (i, 0)),
            pl.BlockSpec((_NT, _LANES), lambda i: (0, 0)),
            pl.BlockSpec((k, h), lambda i: (0, 0)),
            pl.BlockSpec((h, 1), lambda i: (0, 0)),
            pl.BlockSpec((1, 1), lambda i: (0, 0)),
        ],
        out_specs=pl.BlockSpec((_RB, 1), lambda i: (i, 0)),
        out_shape=jax.ShapeDtypeStruct((n, 1), jnp.float32),
    )(p_part, u, part, ws, path_w2, path_b2[None, :])

    return logits2[:, 0]


# speedup vs baseline: 1.0860x; 1.0860x over previous
import functools
import math

import jax
import jax.numpy as jnp
from jax import lax
from jax.experimental import pallas as pl
from jax.experimental.pallas import tpu as pltpu
from jax.experimental.pallas import tpu_sc as plsc

_N = 10000
_D = 384
_DEG = 16
_K = 8
_H = 128
_STEPS = 3

_NT = 16
_CPT = 640
_NP = _NT * _CPT
_CH = 128
_NCH = _CPT // _CH
_LANES = 16
_VPT = _CPT * _K // _LANES

_RB = 2000
_NB = _N // _RB


def _coin_path_body(x_ref, q_ref, cw1_ref, cb1_ref, cw2_ref, cb2_ref,
                    wpe_ref, wpq_ref, pb1_ref, amps_ref, p_ref):
    x = x_ref[...]
    h = jnp.dot(x, cw1_ref[:_D, :], preferred_element_type=jnp.float32)
    hq = jnp.dot(q_ref[...], cw1_ref[_D:, :], preferred_element_type=jnp.float32)
    h = jnp.maximum(h + hq + cb1_ref[...], 0.0)
    amps_ref[...] = (
        jnp.dot(h, cw2_ref[...], preferred_element_type=jnp.float32)
        + cb2_ref[...]
    )
    p = jnp.dot(x, wpe_ref[...], preferred_element_type=jnp.float32)
    pq = jnp.dot(q_ref[...], wpq_ref[...], preferred_element_type=jnp.float32)
    p_ref[...] = p + pq + pb1_ref[...]


def _final_body(p_ref, u_ref, part_ref, ws_ref, pw2_ref, pb2_ref, out_ref):
    ss = jnp.sum(part_ref[...])
    scale = jnp.where(ss > 0.0, lax.rsqrt(ss), 1.0)
    us = jnp.dot(u_ref[...], ws_ref[...], preferred_element_type=jnp.float32)
    hp = jnp.maximum(p_ref[...] + us * scale, 0.0)
    out_ref[...] = (
        jnp.dot(hp, pw2_ref[...], preferred_element_type=jnp.float32)
        + pb2_ref[...]
    )


_S0 = 1.0 / math.sqrt(_N * _K)


def _walk_body(amps_hbm, idx_hbm, u_hbm, part_hbm,
               shared_v, idx_v, amps_v, v_v, acc_v, ss_v, sem):
    t = lax.axis_index("s")
    base = t * _CPT
    last = _NT - 1
    tail = _N - last * _CPT
    iota = lax.iota(jnp.int32, _LANES)
    zeros = jnp.zeros((_LANES,), jnp.float32)

    pltpu.sync_copy(idx_hbm.at[t], idx_v)
    pltpu.sync_copy(amps_hbm.at[pl.ds(base, _CPT)], amps_v)

    @pl.when(t == last)
    def _():
        def zbody(i, carry):
            p = i * _LANES + iota
            plsc.store_scatter(
                amps_v,
                [lax.shift_right_logical(p, 3), lax.bitwise_and(p, 7)],
                zeros,
            )
            return carry
        lax.fori_loop(tail * _K // _LANES, _VPT, zbody, 0)

    def vmul(first):
        def body(i, carry):
            p = i * _LANES + iota
            r = lax.shift_right_logical(p, 3)
            c = lax.bitwise_and(p, 7)
            m = plsc.load_gather(amps_v, [r, c])
            if first:
                val = m * _S0
            else:
                a = plsc.load_gather(acc_v, [r, c])
                val = a * m
            plsc.store_scatter(v_v, [r, c], val)
            plsc.store_scatter(acc_v, [r, c], val)
            return carry
        lax.fori_loop(0, _VPT, body, 0, unroll=4)

    for step in range(_STEPS):
        vmul(step == 0)
        pltpu.sync_copy(v_v, shared_v.at[pl.ds(base, _CPT)])
        plsc.subcore_barrier()
        descs = []
        for g in range(_DEG * _NCH):
            ch = g % _NCH
            descs.append(
                pltpu.async_copy(
                    shared_v.at[idx_v.at[g]],
                    acc_v.at[pl.ds(ch * _CH, _CH)],
                    sem,
                    add=True,
                )
            )
        for d in descs:
            d.wait()
        plsc.subcore_barrier()

    pltpu.sync_copy(acc_v, u_hbm.at[t])

    def ss_body(i, s):
        p = i * _LANES + iota
        a = plsc.load_gather(acc_v, [lax.shift_right_logical(p, 3),
                                     lax.bitwise_and(p, 7)])
        return s + a * a
    ss = lax.fori_loop(0, _VPT, ss_body, jnp.zeros((_LANES,), jnp.float32),
                       unroll=4)
    ss_v[...] = ss
    pltpu.sync_copy(ss_v, part_hbm.at[t])


_walk = functools.partial(
    pl.kernel,
    out_type=(
        jax.ShapeDtypeStruct((_NT, _CPT, _K), jnp.float32),
        jax.ShapeDtypeStruct((_NT, _LANES), jnp.float32),
    ),
    mesh=plsc.VectorSubcoreMesh(
        core_axis_name="c", subcore_axis_name="s", num_cores=1
    ),
    compiler_params=pltpu.CompilerParams(
        needs_layout_passes=False, use_tc_tiling_on_sc=False
    ),
    scratch_types=[
        pltpu.VMEM_SHARED((_NP, _K), jnp.float32),
        pltpu.VMEM((_DEG * _NCH, _CH), jnp.int32),
        pltpu.VMEM((_CPT, _K), jnp.float32),
        pltpu.VMEM((_CPT, _K), jnp.float32),
        pltpu.VMEM((_CPT, _K), jnp.float32),
        pltpu.VMEM((_LANES,), jnp.float32),
        pltpu.SemaphoreType.DMA,
    ],
)(_walk_body)


def kernel(emb_ex, q_emb, neighbors, coin_w1, coin_b1, coin_w2, coin_b2,
           path_w1, path_b1, path_w2, path_b2):
    n, d = emb_ex.shape
    k = coin_w2.shape[1]
    h = coin_w1.shape[1]

    q2 = q_emb[None, :]
    wpe = path_w1[:d]
    ws = path_w1[d:d + k]
    wpq = path_w1[d + k:]

    amps, p_part = pl.pallas_call(
        _coin_path_body,
        grid=(_NB,),
        in_specs=[
            pl.BlockSpec((_RB, d), lambda i: (i, 0)),
            pl.BlockSpec((1, d), lambda i: (0, 0)),
            pl.BlockSpec((2 * d, h), lambda i: (0, 0)),
            pl.BlockSpec((1, h), lambda i: (0, 0)),
            pl.BlockSpec((h, k), lambda i: (0, 0)),
            pl.BlockSpec((1, k), lambda i: (0, 0)),
            pl.BlockSpec((d, h), lambda i: (0, 0)),
            pl.BlockSpec((d, h), lambda i: (0, 0)),
            pl.BlockSpec((1, h), lambda i: (0, 0)),
        ],
        out_specs=[
            pl.BlockSpec((_RB, k), lambda i: (i, 0)),
            pl.BlockSpec((_RB, h), lambda i: (i, 0)),
        ],
        out_shape=[
            jax.ShapeDtypeStruct((_NP, k), jnp.float32),
            jax.ShapeDtypeStruct((n, h), jnp.float32),
        ],
    )(emb_ex, q2, coin_w1, coin_b1[None, :], coin_w2, coin_b2[None, :],
      wpe, wpq, path_b1[None, :])

    nbr = jnp.full((_NP, _DEG), _NP - 1, jnp.int32).at[:n].set(
        neighbors.astype(jnp.int32))
    idx_prep = (
        nbr.reshape(_NT, _NCH, _CH, _DEG)
        .transpose(0, 3, 1, 2)
        .reshape(_NT, _DEG * _NCH, _CH)
    )
    u3, part = _walk(amps, idx_prep)
    u = u3.reshape(_NP, k)

    logits2 = pl.pallas_call(
        _final_body,
        grid=(_NB,),
        in_specs=[
            pl.BlockSpec((_RB, h), lambda i: (i, 0)),
            pl.BlockSpec((_RB, k), lambda i: (i, 0)),
            pl.BlockSpec((_NT, _LANES), lambda i: (0, 0)),
            pl.BlockSpec((k, h), lambda i: (0, 0)),
            pl.BlockSpec((h, 1), lambda i: (0, 0)),
            pl.BlockSpec((1, 1), lambda i: (0, 0)),
        ],
        out_specs=pl.BlockSpec((_RB, 1), lambda i: (i, 0)),
        out_shape=jax.ShapeDtypeStruct((n, 1), jnp.float32),
    )(p_part, u, part, ws, path_w2, path_b2[None, :])

    return logits2[:, 0]


# speedup vs baseline: 1.1388x; 1.0487x over previous
import functools
import math

import jax
import jax.numpy as jnp
from jax import lax
from jax.experimental import pallas as pl
from jax.experimental.pallas import tpu as pltpu
from jax.experimental.pallas import tpu_sc as plsc

_N = 10000
_D = 384
_DEG = 16
_K = 8
_H = 128
_STEPS = 3

_NT = 16
_CPT = 640
_NP = _NT * _CPT
_CH = 128
_NCH = _CPT // _CH
_LANES = 16
_VPT = _CPT * _K // _LANES

_RB = 2000
_NB = _N // _RB


def _coin_body(x_ref, q_ref, cw1_ref, cb1_ref, cw2_ref, cb2_ref, amps_ref):
    x = x_ref[...]
    h = jnp.dot(x, cw1_ref[:_D, :], preferred_element_type=jnp.float32)
    hq = jnp.dot(q_ref[...], cw1_ref[_D:, :], preferred_element_type=jnp.float32)
    h = jnp.maximum(h + hq + cb1_ref[...], 0.0)
    amps_ref[...] = (
        jnp.dot(h, cw2_ref[...], preferred_element_type=jnp.float32)
        + cb2_ref[...]
    )


def _path_pre_body(x_ref, q_ref, wpe_ref, wpq_ref, pb1_ref, p_ref):
    x = x_ref[...]
    p = jnp.dot(x, wpe_ref[...], preferred_element_type=jnp.float32)
    pq = jnp.dot(q_ref[...], wpq_ref[...], preferred_element_type=jnp.float32)
    p_ref[...] = p + pq + pb1_ref[...]


def _final_body(p_ref, u_ref, part_ref, ws_ref, pw2_ref, pb2_ref, out_ref):
    ss = jnp.sum(part_ref[...])
    scale = jnp.where(ss > 0.0, lax.rsqrt(ss), 1.0)
    us = jnp.dot(u_ref[...], ws_ref[...], preferred_element_type=jnp.float32)
    hp = jnp.maximum(p_ref[...] + us * scale, 0.0)
    out_ref[...] = (
        jnp.dot(hp, pw2_ref[...], preferred_element_type=jnp.float32)
        + pb2_ref[...]
    )


_S0 = 1.0 / math.sqrt(_N * _K)


def _walk_body(amps_hbm, idx_hbm, u_hbm, part_hbm,
               shared_v, idx_v, amps_v, v_v, acc_v, ss_v, sem):
    t = lax.axis_index("s")
    base = t * _CPT
    last = _NT - 1
    tail = _N - last * _CPT
    iota = lax.iota(jnp.int32, _LANES)
    zeros = jnp.zeros((_LANES,), jnp.float32)

    pltpu.sync_copy(idx_hbm.at[t], idx_v)
    pltpu.sync_copy(amps_hbm.at[pl.ds(base, _CPT)], amps_v)

    @pl.when(t == last)
    def _():
        def zbody(i, carry):
            p = i * _LANES + iota
            plsc.store_scatter(
                amps_v,
                [lax.shift_right_logical(p, 3), lax.bitwise_and(p, 7)],
                zeros,
            )
            return carry
        lax.fori_loop(tail * _K // _LANES, _VPT, zbody, 0)

    def vmul(first):
        def body(i, carry):
            p = i * _LANES + iota
            r = lax.shift_right_logical(p, 3)
            c = lax.bitwise_and(p, 7)
            m = plsc.load_gather(amps_v, [r, c])
            if first:
                val = m * _S0
            else:
                a = plsc.load_gather(acc_v, [r, c])
                val = a * m
            plsc.store_scatter(v_v, [r, c], val)
            plsc.store_scatter(acc_v, [r, c], val)
            return carry
        lax.fori_loop(0, _VPT, body, 0, unroll=4)

    for step in range(_STEPS):
        vmul(step == 0)
        pltpu.sync_copy(v_v, shared_v.at[pl.ds(base, _CPT)])
        plsc.subcore_barrier()
        descs = []
        for g in range(_DEG * _NCH):
            ch = g % _NCH
            descs.append(
                pltpu.async_copy(
                    shared_v.at[idx_v.at[g]],
                    acc_v.at[pl.ds(ch * _CH, _CH)],
                    sem,
                    add=True,
                )
            )
        for d in descs:
            d.wait()
        plsc.subcore_barrier()

    pltpu.sync_copy(acc_v, u_hbm.at[t])

    def ss_body(i, s):
        p = i * _LANES + iota
        a = plsc.load_gather(acc_v, [lax.shift_right_logical(p, 3),
                                     lax.bitwise_and(p, 7)])
        return s + a * a
    ss = lax.fori_loop(0, _VPT, ss_body, jnp.zeros((_LANES,), jnp.float32),
                       unroll=4)
    ss_v[...] = ss
    pltpu.sync_copy(ss_v, part_hbm.at[t])


_walk = functools.partial(
    pl.kernel,
    out_type=(
        jax.ShapeDtypeStruct((_NT, _CPT, _K), jnp.float32),
        jax.ShapeDtypeStruct((_NT, _LANES), jnp.float32),
    ),
    mesh=plsc.VectorSubcoreMesh(
        core_axis_name="c", subcore_axis_name="s", num_cores=1
    ),
    compiler_params=pltpu.CompilerParams(
        needs_layout_passes=False, use_tc_tiling_on_sc=False
    ),
    scratch_types=[
        pltpu.VMEM_SHARED((_NP, _K), jnp.float32),
        pltpu.VMEM((_DEG * _NCH, _CH), jnp.int32),
        pltpu.VMEM((_CPT, _K), jnp.float32),
        pltpu.VMEM((_CPT, _K), jnp.float32),
        pltpu.VMEM((_CPT, _K), jnp.float32),
        pltpu.VMEM((_LANES,), jnp.float32),
        pltpu.SemaphoreType.DMA,
    ],
)(_walk_body)


def kernel(emb_ex, q_emb, neighbors, coin_w1, coin_b1, coin_w2, coin_b2,
           path_w1, path_b1, path_w2, path_b2):
    n, d = emb_ex.shape
    k = coin_w2.shape[1]
    h = coin_w1.shape[1]

    q2 = q_emb[None, :]
    wpe = path_w1[:d]
    ws = path_w1[d:d + k]
    wpq = path_w1[d + k:]

    amps = pl.pallas_call(
        _coin_body,
        grid=(_NB,),
        in_specs=[
            pl.BlockSpec((_RB, d), lambda i: (i, 0)),
            pl.BlockSpec((1, d), lambda i: (0, 0)),
            pl.BlockSpec((2 * d, h), lambda i: (0, 0)),
            pl.BlockSpec((1, h), lambda i: (0, 0)),
            pl.BlockSpec((h, k), lambda i: (0, 0)),
            pl.BlockSpec((1, k), lambda i: (0, 0)),
        ],
        out_specs=pl.BlockSpec((_RB, k), lambda i: (i, 0)),
        out_shape=jax.ShapeDtypeStruct((_NP, k), jnp.float32),
    )(emb_ex, q2, coin_w1, coin_b1[None, :], coin_w2, coin_b2[None, :])

    p_part = pl.pallas_call(
        _path_pre_body,
        grid=(_NB,),
        in_specs=[
            pl.BlockSpec((_RB, d), lambda i: (i, 0)),
            pl.BlockSpec((1, d), lambda i: (0, 0)),
            pl.BlockSpec((d, h), lambda i: (0, 0)),
            pl.BlockSpec((d, h), lambda i: (0, 0)),
            pl.BlockSpec((1, h), lambda i: (0, 0)),
        ],
        out_specs=pl.BlockSpec((_RB, h), lambda i: (i, 0)),
        out_shape=jax.ShapeDtypeStruct((n, h), jnp.float32),
    )(emb_ex, q2, wpe, wpq, path_b1[None, :])

    nbr = jnp.full((_NP, _DEG), _NP - 1, jnp.int32).at[:n].set(
        neighbors.astype(jnp.int32))
    idx_prep = (
        nbr.reshape(_NT, _NCH, _CH, _DEG)
        .transpose(0, 3, 1, 2)
        .reshape(_NT, _DEG * _NCH, _CH)
    )
    u3, part = _walk(amps, idx_prep)
    u = u3.reshape(_NP, k)

    logits2 = pl.pallas_call(
        _final_body,
        grid=(_NB,),
        in_specs=[
            pl.BlockSpec((_RB, h), lambda i: (i, 0)),
            pl.BlockSpec((_RB, k), lambda i: (i, 0)),
            pl.BlockSpec((_NT, _LANES), lambda i: (0, 0)),
            pl.BlockSpec((k, h), lambda i: (0, 0)),
            pl.BlockSpec((h, 1), lambda i: (0, 0)),
            pl.BlockSpec((1, 1), lambda i: (0, 0)),
        ],
        out_specs=pl.BlockSpec((_RB, 1), lambda i: (i, 0)),
        out_shape=jax.ShapeDtypeStruct((n, 1), jnp.float32),
    )(p_part, u, part, ws, path_w2, path_b2[None, :])

    return logits2[:, 0]


# speedup vs baseline: 1.1636x; 1.0217x over previous
import functools
import math

import jax
import jax.numpy as jnp
from jax import lax
from jax.experimental import pallas as pl
from jax.experimental.pallas import tpu as pltpu
from jax.experimental.pallas import tpu_sc as plsc

_N = 10000
_D = 384
_DEG = 16
_K = 8
_H = 128
_STEPS = 3

_NT = 16
_CPT = 640
_NP = _NT * _CPT
_CH = 128
_NCH = _CPT // _CH
_LANES = 16
_VPT = _CPT * _K // _LANES

_RB = 2000
_NB = _N // _RB


def _coin_body(x_ref, q_ref, cw1_ref, cb1_ref, cw2_ref, cb2_ref, amps_ref):
    x = x_ref[...]
    h = jnp.dot(x, cw1_ref[:_D, :], preferred_element_type=jnp.float32)
    hq = jnp.dot(q_ref[...], cw1_ref[_D:, :], preferred_element_type=jnp.float32)
    h = jnp.maximum(h + hq + cb1_ref[...], 0.0)
    amps_ref[...] = (
        jnp.dot(h, cw2_ref[...], preferred_element_type=jnp.float32)
        + cb2_ref[...]
    )


def _path_pre_body(x_ref, q_ref, wpe_ref, wpq_ref, pb1_ref, p_ref):
    x = x_ref[...]
    p = jnp.dot(x, wpe_ref[...], preferred_element_type=jnp.float32)
    pq = jnp.dot(q_ref[...], wpq_ref[...], preferred_element_type=jnp.float32)
    p_ref[...] = p + pq + pb1_ref[...]


def _final_body(p_ref, u_ref, part_ref, ws_ref, pw2_ref, pb2_ref, out_ref):
    ss = jnp.sum(part_ref[...])
    scale = jnp.where(ss > 0.0, lax.rsqrt(ss), 1.0)
    us = jnp.dot(u_ref[...], ws_ref[...], preferred_element_type=jnp.float32)
    hp = jnp.maximum(p_ref[...] + us * scale, 0.0)
    out_ref[...] = (
        jnp.dot(hp, pw2_ref[...], preferred_element_type=jnp.float32)
        + pb2_ref[...]
    )


_S0 = 1.0 / math.sqrt(_N * _K)


def _walk_body(amps_hbm, idx_hbm, u_hbm, part_hbm,
               shared_a, shared_b, idx_v, amps_v, v_v, acc_v, ss_v, sem, sem2):
    t = lax.axis_index("s")
    base = t * _CPT
    last = _NT - 1
    tail = _N - last * _CPT
    iota = lax.iota(jnp.int32, _LANES)
    zeros = jnp.zeros((_LANES,), jnp.float32)

    idx_dma = pltpu.async_copy(idx_hbm.at[t], idx_v, sem2)
    pltpu.sync_copy(amps_hbm.at[pl.ds(base, _CPT)], amps_v)

    @pl.when(t == last)
    def _():
        def zbody(i, carry):
            p = i * _LANES + iota
            plsc.store_scatter(
                amps_v,
                [lax.shift_right_logical(p, 3), lax.bitwise_and(p, 7)],
                zeros,
            )
            return carry
        lax.fori_loop(tail * _K // _LANES, _VPT, zbody, 0)

    def vmul(first):
        def body(i, carry):
            p = i * _LANES + iota
            r = lax.shift_right_logical(p, 3)
            c = lax.bitwise_and(p, 7)
            m = plsc.load_gather(amps_v, [r, c])
            if first:
                val = m * _S0
            else:
                a = plsc.load_gather(acc_v, [r, c])
                val = a * m
            plsc.store_scatter(v_v, [r, c], val)
            plsc.store_scatter(acc_v, [r, c], val)
            return carry
        lax.fori_loop(0, _VPT, body, 0, unroll=4)

    for step in range(_STEPS):
        vmul(step == 0)
        table = shared_a if step % 2 == 0 else shared_b
        pltpu.sync_copy(v_v, table.at[pl.ds(base, _CPT)])
        plsc.subcore_barrier()
        if step == 0:
            idx_dma.wait()
        descs = []
        for g in range(_DEG * _NCH):
            ch = g % _NCH
            descs.append(
                pltpu.async_copy(
                    table.at[idx_v.at[g]],
                    acc_v.at[pl.ds(ch * _CH, _CH)],
                    sem,
                    add=True,
                )
            )
        for d in descs:
            d.wait()

    u_dma = pltpu.async_copy(acc_v, u_hbm.at[t], sem2)

    def ss_body(i, s):
        p = i * _LANES + iota
        a = plsc.load_gather(acc_v, [lax.shift_right_logical(p, 3),
                                     lax.bitwise_and(p, 7)])
        return s + a * a
    ss = lax.fori_loop(0, _VPT, ss_body, jnp.zeros((_LANES,), jnp.float32),
                       unroll=4)
    ss_v[...] = ss
    pltpu.sync_copy(ss_v, part_hbm.at[t])
    u_dma.wait()


_walk = functools.partial(
    pl.kernel,
    out_type=(
        jax.ShapeDtypeStruct((_NT, _CPT, _K), jnp.float32),
        jax.ShapeDtypeStruct((_NT, _LANES), jnp.float32),
    ),
    mesh=plsc.VectorSubcoreMesh(
        core_axis_name="c", subcore_axis_name="s", num_cores=1
    ),
    compiler_params=pltpu.CompilerParams(
        needs_layout_passes=False, use_tc_tiling_on_sc=False
    ),
    scratch_types=[
        pltpu.VMEM_SHARED((_NP, _K), jnp.float32),
        pltpu.VMEM_SHARED((_NP, _K), jnp.float32),
        pltpu.VMEM((_DEG * _NCH, _CH), jnp.int32),
        pltpu.VMEM((_CPT, _K), jnp.float32),
        pltpu.VMEM((_CPT, _K), jnp.float32),
        pltpu.VMEM((_CPT, _K), jnp.float32),
        pltpu.VMEM((_LANES,), jnp.float32),
        pltpu.SemaphoreType.DMA,
        pltpu.SemaphoreType.DMA,
    ],
)(_walk_body)


def kernel(emb_ex, q_emb, neighbors, coin_w1, coin_b1, coin_w2, coin_b2,
           path_w1, path_b1, path_w2, path_b2):
    n, d = emb_ex.shape
    k = coin_w2.shape[1]
    h = coin_w1.shape[1]

    q2 = q_emb[None, :]
    wpe = path_w1[:d]
    ws = path_w1[d:d + k]
    wpq = path_w1[d + k:]

    amps = pl.pallas_call(
        _coin_body,
        grid=(_NB,),
        in_specs=[
            pl.BlockSpec((_RB, d), lambda i: (i, 0)),
            pl.BlockSpec((1, d), lambda i: (0, 0)),
            pl.BlockSpec((2 * d, h), lambda i: (0, 0)),
            pl.BlockSpec((1, h), lambda i: (0, 0)),
            pl.BlockSpec((h, k), lambda i: (0, 0)),
            pl.BlockSpec((1, k), lambda i: (0, 0)),
        ],
        out_specs=pl.BlockSpec((_RB, k), lambda i: (i, 0)),
        out_shape=jax.ShapeDtypeStruct((_NP, k), jnp.float32),
    )(emb_ex, q2, coin_w1, coin_b1[None, :], coin_w2, coin_b2[None, :])

    p_part = pl.pallas_call(
        _path_pre_body,
        grid=(_NB,),
        in_specs=[
            pl.BlockSpec((_RB, d), lambda i: (i, 0)),
            pl.BlockSpec((1, d), lambda i: (0, 0)),
            pl.BlockSpec((d, h), lambda i: (0, 0)),
            pl.BlockSpec((d, h), lambda i: (0, 0)),
            pl.BlockSpec((1, h), lambda i: (0, 0)),
        ],
        out_specs=pl.BlockSpec((_RB, h), lambda i: (i, 0)),
        out_shape=jax.ShapeDtypeStruct((n, h), jnp.float32),
    )(emb_ex, q2, wpe, wpq, path_b1[None, :])

    nbr = jnp.full((_NP, _DEG), _NP - 1, jnp.int32).at[:n].set(
        neighbors.astype(jnp.int32))
    idx_prep = (
        nbr.reshape(_NT, _NCH, _CH, _DEG)
        .transpose(0, 3, 1, 2)
        .reshape(_NT, _DEG * _NCH, _CH)
    )
    u3, part = _walk(amps, idx_prep)
    u = u3.reshape(_NP, k)

    logits2 = pl.pallas_call(
        _final_body,
        grid=(_NB,),
        in_specs=[
            pl.BlockSpec((_RB, h), lambda i: (i, 0)),
            pl.BlockSpec((_RB, k), lambda i: (i, 0)),
            pl.BlockSpec((_NT, _LANES), lambda i: (0, 0)),
            pl.BlockSpec((k, h), lambda i: (0, 0)),
            pl.BlockSpec((h, 1), lambda i: (0, 0)),
            pl.BlockSpec((1, 1), lambda i: (0, 0)),
        ],
        out_specs=pl.BlockSpec((_RB, 1), lambda i: (i, 0)),
        out_shape=jax.ShapeDtypeStruct((n, 1), jnp.float32),
    )(p_part, u, part, ws, path_w2, path_b2[None, :])

    return logits2[:, 0]
